# Initial kernel scaffold; baseline (speedup 1.0000x reference)
import jax
import jax.numpy as jnp
from jax.experimental import pallas as pl


def kernel(image, perm):
    raise NotImplementedError("write your pallas kernel here")


# speedup vs baseline: 1.0390x; 1.0390x over previous
import dataclasses
import functools

import jax
import jax.numpy as jnp
from jax import lax
from jax.experimental import pallas as pl
from jax.experimental.pallas import tpu as pltpu
from jax.experimental.pallas import tpu_sc as plsc


def kernel(image, perm):
    C, W, H = image.shape
    P = W * H
    L = 16
    NC, NS = 2, 16
    NW = NC * NS
    BP = 128

    assert C % L == 0 and P % (NW * BP) == 0
    blocks_per_worker = P // (NW * BP)

    img2 = image.reshape(C, P)
    perm2 = perm.reshape(P, C)

    mesh = plsc.VectorSubcoreMesh(core_axis_name="c", subcore_axis_name="s",
                                  num_cores=NC, num_subcores=NS)

    cp = pltpu.CompilerParams()
    if "needs_layout_passes" in pltpu.CompilerParams.__dataclass_fields__:
        cp = dataclasses.replace(cp, needs_layout_passes=False)

    @functools.partial(
        pl.kernel,
        compiler_params=cp,
        out_type=jax.ShapeDtypeStruct((C, P), jnp.float32),
        mesh=mesh,
        scratch_types=[
            pltpu.VMEM((C, BP), jnp.float32),
            pltpu.VMEM((BP, C), jnp.int32),
            pltpu.VMEM((C, BP), jnp.float32),
        ],
    )
    def permute_kernel(img_hbm, perm_hbm, out_hbm, img_v, perm_v, out_v):
        wid = lax.axis_index("s") * NC + lax.axis_index("c")
        rows = [c0 + lax.iota(jnp.int32, L) for c0 in range(0, C, L)]

        @pl.loop(0, blocks_per_worker)
        def _block(b):
            p0 = (wid * blocks_per_worker + b) * BP
            pltpu.sync_copy(img_hbm.at[:, pl.ds(p0, BP)], img_v)
            pltpu.sync_copy(perm_hbm.at[pl.ds(p0, BP), :], perm_v)

            @pl.loop(0, BP)
            def _pixel(p):
                sp = jnp.full((L,), p, jnp.int32)
                for k in range(C // L):
                    pv = perm_v[p, pl.ds(k * L, L)]
                    vals = plsc.load_gather(img_v, [pv, sp])
                    plsc.store_scatter(out_v, [rows[k], sp], vals)

            pltpu.sync_copy(out_v, out_hbm.at[:, pl.ds(p0, BP)])

    out2 = permute_kernel(img2, perm2)
    return out2.reshape(C, W, H)


# speedup vs baseline: 2.2910x; 2.2050x over previous
import dataclasses
import functools

import jax
import jax.numpy as jnp
from jax import lax
from jax.experimental import pallas as pl
from jax.experimental.pallas import tpu as pltpu
from jax.experimental.pallas import tpu_sc as plsc


def kernel(image, perm):
    C, W, H = image.shape
    P = W * H
    L = 16
    NC, NS = 2, 16
    NW = NC * NS
    BP = 128

    assert C % L == 0 and P % (NW * BP) == 0
    blocks_per_worker = P // (NW * BP)

    img2 = image.reshape(C, P)
    perm2 = perm.reshape(P, C)

    mesh = plsc.VectorSubcoreMesh(core_axis_name="c", subcore_axis_name="s",
                                  num_cores=NC, num_subcores=NS)

    cp = pltpu.CompilerParams()
    if "needs_layout_passes" in pltpu.CompilerParams.__dataclass_fields__:
        cp = dataclasses.replace(cp, needs_layout_passes=False)

    @functools.partial(
        pl.kernel,
        compiler_params=cp,
        out_type=jax.ShapeDtypeStruct((C, P), jnp.float32),
        mesh=mesh,
        scratch_types=[
            pltpu.VMEM((C, BP), jnp.float32),
            pltpu.VMEM((BP, C), jnp.int32),
            pltpu.VMEM((C, BP), jnp.float32),
        ],
    )
    def permute_kernel(img_hbm, perm_hbm, out_hbm, img_v, perm_v, out_v):
        wid = lax.axis_index("s") * NC + lax.axis_index("c")
        iot = lax.iota(jnp.int32, L)
        iotaqs = [q0 + iot for q0 in range(0, BP, L)]

        @pl.loop(0, blocks_per_worker)
        def _block(b):
            p0 = (wid * blocks_per_worker + b) * BP
            pltpu.sync_copy(img_hbm.at[:, pl.ds(p0, BP)], img_v)
            pltpu.sync_copy(perm_hbm.at[pl.ds(p0, BP), :], perm_v)

            @plsc.parallel_loop(0, C, unroll=2)
            def _chan(c):
                sc = jnp.full((L,), c, jnp.int32)
                for qi in range(BP // L):
                    pv = plsc.load_gather(perm_v, [iotaqs[qi], sc])
                    vals = plsc.load_gather(img_v, [pv, iotaqs[qi]])
                    out_v[c, pl.ds(qi * L, L)] = vals

            pltpu.sync_copy(out_v, out_hbm.at[:, pl.ds(p0, BP)])

    out2 = permute_kernel(img2, perm2)
    return out2.reshape(C, W, H)
